# baseline (speedup 1.0000x reference)
import functools

import jax
import jax.numpy as jnp
from jax import lax
from jax.experimental import pallas as pl
from jax.experimental.pallas import tpu as pltpu
from jax.experimental.pallas import tpu_sc as plsc

D = 64
NC, NS = 2, 16
NW = NC * NS
IDX_W = 128
CHUNK = 640
G = CHUNK // IDX_W


def _make_kernel(n_rows: int, vocab: int):
    rows_per_w = n_rows // NW
    n_chunks = rows_per_w // CHUNK
    idx_rows = rows_per_w // IDX_W

    mesh = plsc.VectorSubcoreMesh(core_axis_name="c", subcore_axis_name="s")

    @functools.partial(
        pl.kernel,
        mesh=mesh,
        compiler_params=pltpu.CompilerParams(use_tc_tiling_on_sc=False),
        out_type=jax.ShapeDtypeStruct((n_rows, 2 * D), jnp.float32),
        scratch_types=[
            pltpu.VMEM((idx_rows, IDX_W), jnp.int32),
            pltpu.VMEM((CHUNK, D), jnp.float32),
            pltpu.VMEM((CHUNK, D), jnp.float32),
            pltpu.SemaphoreType.DMA,
            pltpu.SemaphoreType.DMA,
            pltpu.SemaphoreType.DMA,
            pltpu.SemaphoreType.DMA,
            pltpu.SemaphoreType.DMA,
        ],
    )
    def k(idx_hbm, other_hbm, table_hbm, out_hbm,
          idx_v, buf0, buf1, gsem0, gsem1, wsem0, wsem1, dsem):
        wid = lax.axis_index("s") * NC + lax.axis_index("c")
        base_w = wid * rows_per_w
        bufs = (buf0, buf1)
        gsems = (gsem0, gsem1)
        wsems = (wsem0, wsem1)
        pltpu.sync_copy(idx_hbm.at[wid], idx_v)

        dense = []
        if True:
            for c in range(n_chunks):
                base = base_w + c * CHUNK
                dense.append(pltpu.async_copy(
                    other_hbm.at[pl.ds(base, CHUNK)],
                    out_hbm.at[pl.ds(base, CHUNK), pl.ds(D, D)],
                    dsem,
                ))

        def fire(c):
            b = c % 2
            return [pltpu.async_copy(
                table_hbm.at[idx_v.at[c * G + g]],
                bufs[b].at[pl.ds(g * IDX_W, IDX_W)],
                gsems[b],
            ) for g in range(G)]

        if False:
            pending_w = [None, None]
            gh = {0: fire(0)}
            for c in range(n_chunks):
                if c + 1 < n_chunks:
                    b = (c + 1) % 2
                    if pending_w[b] is not None:
                        pending_w[b].wait()
                        pending_w[b] = None
                    gh[c + 1] = fire(c + 1)
                for h in gh.pop(c):
                    h.wait()
                base = base_w + c * CHUNK
                pending_w[c % 2] = pltpu.async_copy(
                    bufs[c % 2],
                    out_hbm.at[pl.ds(base, CHUNK), pl.ds(0, D)],
                    wsems[c % 2],
                )
            for w in pending_w:
                if w is not None:
                    w.wait()
        for cp in dense:
            cp.wait()

    return k


def kernel(indices, other_features, table):
    b, l = indices.shape
    n_rows = b * l
    vocab = table.shape[0]
    idx_r = indices.reshape(NW, n_rows // (NW * IDX_W), IDX_W).astype(jnp.int32)
    other_r = other_features.reshape(n_rows, D)
    out = _make_kernel(n_rows, vocab)(idx_r, other_r, table)
    return out.reshape(b, l, 2 * D)


# speedup vs baseline: 4.2793x; 4.2793x over previous
import functools

import jax
import jax.numpy as jnp
from jax import lax
from jax.experimental import pallas as pl
from jax.experimental.pallas import tpu as pltpu
from jax.experimental.pallas import tpu_sc as plsc

D = 64
NC, NS = 2, 16
NW = NC * NS
IDX_W = 128
CHUNK = 256
G = CHUNK // IDX_W


def _make_kernel(n_rows: int):
    rows_per_w = n_rows // NW
    n_chunks = rows_per_w // CHUNK
    idx_rows = rows_per_w // IDX_W

    mesh = plsc.VectorSubcoreMesh(core_axis_name="c", subcore_axis_name="s")

    @functools.partial(
        pl.kernel,
        mesh=mesh,
        compiler_params=pltpu.CompilerParams(use_tc_tiling_on_sc=False),
        out_type=jax.ShapeDtypeStruct((n_rows, 2 * D), jnp.float32),
        scratch_types=[
            pltpu.VMEM((idx_rows, IDX_W), jnp.int32),
            pltpu.VMEM((CHUNK, D), jnp.float32),
            pltpu.VMEM((CHUNK, D), jnp.float32),
            pltpu.VMEM((CHUNK, D), jnp.float32),
            pltpu.VMEM((CHUNK, D), jnp.float32),
            pltpu.SemaphoreType.DMA,
            pltpu.SemaphoreType.DMA,
            pltpu.SemaphoreType.DMA,
            pltpu.SemaphoreType.DMA,
            pltpu.SemaphoreType.DMA,
            pltpu.SemaphoreType.DMA,
            pltpu.SemaphoreType.DMA,
            pltpu.SemaphoreType.DMA,
        ],
    )
    def k(idx_hbm, other_hbm, table_hbm, out_hbm, idx_v,
          gb0, gb1, fb0, fb1, gl0, gl1, fl0, fl1, gw0, gw1, fw0, fw1):
        wid = lax.axis_index("s") * NC + lax.axis_index("c")
        base_w = wid * rows_per_w
        gbufs, fbufs = (gb0, gb1), (fb0, fb1)
        glsem, flsem = (gl0, gl1), (fl0, fl1)
        gwsem, fwsem = (gw0, gw1), (fw0, fw1)
        pltpu.sync_copy(idx_hbm.at[wid], idx_v)

        gload = [None, None]
        fload = [None, None]
        gwrite = [None, None]
        fwrite = [None, None]

        def load(c):
            b = c % 2
            base = base_w + c * CHUNK
            gload[b] = [pltpu.async_copy(
                table_hbm.at[idx_v.at[c * G + g]],
                gbufs[b].at[pl.ds(g * IDX_W, IDX_W)],
                glsem[b],
            ) for g in range(G)]
            fload[b] = pltpu.async_copy(
                other_hbm.at[pl.ds(base, CHUNK)], fbufs[b], flsem[b])

        load(0)
        for c in range(n_chunks):
            b = c % 2
            if c + 1 < n_chunks:
                nb = (c + 1) % 2
                if gwrite[nb] is not None:
                    gwrite[nb].wait()
                    fwrite[nb].wait()
                load(c + 1)
            base = base_w + c * CHUNK
            for h in gload[b]:
                h.wait()
            gwrite[b] = pltpu.async_copy(
                gbufs[b], out_hbm.at[pl.ds(base, CHUNK), pl.ds(0, D)],
                gwsem[b])
            fload[b].wait()
            fwrite[b] = pltpu.async_copy(
                fbufs[b], out_hbm.at[pl.ds(base, CHUNK), pl.ds(D, D)],
                fwsem[b])
        for w in gwrite + fwrite:
            if w is not None:
                w.wait()

    return k


def kernel(indices, other_features, table):
    b, l = indices.shape
    n_rows = b * l
    idx_r = indices.reshape(NW, n_rows // (NW * IDX_W), IDX_W).astype(jnp.int32)
    other_r = other_features.reshape(n_rows, D)
    out = _make_kernel(n_rows)(idx_r, other_r, table)
    return out.reshape(b, l, 2 * D)


# speedup vs baseline: 4.4513x; 1.0402x over previous
import functools

import jax
import jax.numpy as jnp
from jax import lax
from jax.experimental import pallas as pl
from jax.experimental.pallas import tpu as pltpu
from jax.experimental.pallas import tpu_sc as plsc

D = 64
NC, NS = 2, 16
NW = NC * NS
IDX_W = 128
CHUNK = 640
G = CHUNK // IDX_W


def _make_gather(n_rows: int):
    rows_per_w = n_rows // NW
    n_chunks = rows_per_w // CHUNK
    idx_rows = rows_per_w // IDX_W

    mesh = plsc.VectorSubcoreMesh(core_axis_name="c", subcore_axis_name="s")

    @functools.partial(
        pl.kernel,
        mesh=mesh,
        compiler_params=pltpu.CompilerParams(use_tc_tiling_on_sc=False),
        out_type=jax.ShapeDtypeStruct((n_rows, D), jnp.float32),
        scratch_types=[
            pltpu.VMEM((idx_rows, IDX_W), jnp.int32),
            pltpu.VMEM((CHUNK, D), jnp.float32),
            pltpu.VMEM((CHUNK, D), jnp.float32),
            pltpu.SemaphoreType.DMA,
            pltpu.SemaphoreType.DMA,
            pltpu.SemaphoreType.DMA,
            pltpu.SemaphoreType.DMA,
        ],
    )
    def k(idx_hbm, table_hbm, out_hbm, idx_v, b0, b1, g0, g1, w0, w1):
        wid = lax.axis_index("s") * NC + lax.axis_index("c")
        base_w = wid * rows_per_w
        bufs, gsems, wsems = (b0, b1), (g0, g1), (w0, w1)
        pltpu.sync_copy(idx_hbm.at[pl.ds(wid * idx_rows, idx_rows)], idx_v)

        gload = [None, None]
        write = [None, None]

        def fire(c):
            b = c % 2
            gload[b] = [pltpu.async_copy(
                table_hbm.at[idx_v.at[c * G + g]],
                bufs[b].at[pl.ds(g * IDX_W, IDX_W)],
                gsems[b],
            ) for g in range(G)]

        fire(0)
        for c in range(n_chunks):
            b = c % 2
            if c + 1 < n_chunks:
                nb = (c + 1) % 2
                if write[nb] is not None:
                    write[nb].wait()
                fire(c + 1)
            for h in gload[b]:
                h.wait()
            write[b] = pltpu.async_copy(
                bufs[b],
                out_hbm.at[pl.ds(base_w + c * CHUNK, CHUNK)],
                wsems[b])
        for w in write:
            if w is not None:
                w.wait()

    return k


def _concat_body(gath_ref, other_ref, out_ref):
    bm, l, _ = out_ref.shape
    a = gath_ref[:, :D]
    b = gath_ref[:, D:]
    e = jnp.stack([a, b], axis=1).reshape(bm, l, D)
    out_ref[:, :, :D] = e
    out_ref[:, :, D:] = other_ref[...]


def _make_concat(b: int, l: int, bm: int):
    grid = (b // bm,)
    return pl.pallas_call(
        _concat_body,
        grid=grid,
        in_specs=[
            pl.BlockSpec((bm * l // 2, 2 * D), lambda i: (i, 0)),
            pl.BlockSpec((bm, l, D), lambda i: (i, 0, 0)),
        ],
        out_specs=pl.BlockSpec((bm, l, 2 * D), lambda i: (i, 0, 0)),
        out_shape=jax.ShapeDtypeStruct((b, l, 2 * D), jnp.float32),
    )


def kernel(indices, other_features, table):
    b, l = indices.shape
    n_rows = b * l
    idx_r = indices.reshape(n_rows // IDX_W, IDX_W).astype(jnp.int32)
    gath = _make_gather(n_rows)(idx_r, table)
    gath2 = gath.reshape(n_rows // 2, 2 * D)
    return _make_concat(b, l, 32)(gath2, other_features)


# speedup vs baseline: 5.3191x; 1.1950x over previous
import functools

import jax
import jax.numpy as jnp
from jax import lax
from jax.experimental import pallas as pl
from jax.experimental.pallas import tpu as pltpu
from jax.experimental.pallas import tpu_sc as plsc

D = 64
NC, NS = 2, 16
NW = NC * NS
LP = 56
NB = 8


def _make_gather(b: int, l: int, vocab: int):
    b_per_w = b // NW
    n_chunks = b_per_w // NB

    mesh = plsc.VectorSubcoreMesh(core_axis_name="c", subcore_axis_name="s")

    rows_per_c = NB * l
    GW = 80
    n_g = rows_per_c // GW

    @functools.partial(
        pl.kernel,
        mesh=mesh,
        compiler_params=pltpu.CompilerParams(use_tc_tiling_on_sc=False),
        out_type=jax.ShapeDtypeStruct((b, LP, 2 * D), jnp.float32),
        scratch_types=[
            pltpu.VMEM((b_per_w * l,), jnp.int32),
            pltpu.VMEM((rows_per_c, D), jnp.float32),
            pltpu.VMEM((rows_per_c, D), jnp.float32),
            pltpu.SemaphoreType.DMA,
            pltpu.SemaphoreType.DMA,
            pltpu.SemaphoreType.DMA,
            pltpu.SemaphoreType.DMA,
        ],
    )
    def k(idx_hbm, table_hbm, out_hbm, idx_v, b0, b1, g0, g1, w0, w1):
        wid = lax.axis_index("s") * NC + lax.axis_index("c")
        base_w = wid * b_per_w
        bufs, gsems, wsems = (b0, b1), (g0, g1), (w0, w1)
        pltpu.sync_copy(idx_hbm.at[pl.ds(base_w * l, b_per_w * l)], idx_v)

        gload = [None, None]
        write = [[], []]

        def fire(c):
            bb = c % 2
            gload[bb] = [pltpu.async_copy(
                table_hbm.at[idx_v.at[pl.ds(c * rows_per_c + g * GW, GW)]],
                bufs[bb].at[pl.ds(g * GW, GW)],
                gsems[bb],
            ) for g in range(n_g)]

        fire(0)
        for c in range(n_chunks):
            bb = c % 2
            if c + 1 < n_chunks:
                nb = (c + 1) % 2
                for w in write[nb]:
                    w.wait()
                write[nb] = []
                fire(c + 1)
            for h in gload[bb]:
                h.wait()
            write[bb] = [pltpu.async_copy(
                bufs[bb].at[pl.ds(j * l, l)],
                out_hbm.at[base_w + c * NB + j].at[pl.ds(0, l), pl.ds(0, D)],
                wsems[bb],
            ) for j in range(NB)]
        for ws in write:
            for w in ws:
                w.wait()

    return k


def _concat_body(gath_ref, other_ref, out_ref):
    bm, l, _ = out_ref.shape
    out_ref[:, :, :D] = gath_ref[:, :l, :D]
    out_ref[:, :, D:] = other_ref[...]


def _make_concat(b: int, l: int, bm: int):
    return pl.pallas_call(
        _concat_body,
        grid=(b // bm,),
        in_specs=[
            pl.BlockSpec((bm, LP, 2 * D), lambda i: (i, 0, 0)),
            pl.BlockSpec((bm, l, D), lambda i: (i, 0, 0)),
        ],
        out_specs=pl.BlockSpec((bm, l, 2 * D), lambda i: (i, 0, 0)),
        out_shape=jax.ShapeDtypeStruct((b, l, 2 * D), jnp.float32),
    )


def kernel(indices, other_features, table):
    b, l = indices.shape
    vocab = table.shape[0]
    idx_i32 = indices.reshape(b * l).astype(jnp.int32)
    gath = _make_gather(b, l, vocab)(idx_i32, table)
    return _make_concat(b, l, 64)(gath, other_features)


# speedup vs baseline: 5.3591x; 1.0075x over previous
import functools

import jax
import jax.numpy as jnp
from jax import lax
from jax.experimental import pallas as pl
from jax.experimental.pallas import tpu as pltpu
from jax.experimental.pallas import tpu_sc as plsc

D = 64
NC, NS = 2, 16
NW = NC * NS
NB = 8
GW = 80


def _make_gather(b: int, l: int):
    b_per_w = b // NW
    n_chunks = b_per_w // NB
    rows_per_c = NB * l
    n_g = rows_per_c // GW

    mesh = plsc.VectorSubcoreMesh(core_axis_name="c", subcore_axis_name="s")

    @functools.partial(
        pl.kernel,
        mesh=mesh,
        compiler_params=pltpu.CompilerParams(use_tc_tiling_on_sc=False),
        out_type=jax.ShapeDtypeStruct((b * l, 2 * D), jnp.float32),
        scratch_types=[
            pltpu.VMEM((b_per_w * l,), jnp.int32),
            pltpu.VMEM((rows_per_c, D), jnp.float32),
            pltpu.VMEM((rows_per_c, D), jnp.float32),
            pltpu.SemaphoreType.DMA,
            pltpu.SemaphoreType.DMA,
            pltpu.SemaphoreType.DMA,
            pltpu.SemaphoreType.DMA,
        ],
    )
    def k(idx_hbm, table_hbm, out_hbm, idx_v, b0, b1, g0, g1, w0, w1):
        wid = lax.axis_index("s") * NC + lax.axis_index("c")
        base_w = wid * b_per_w
        bufs, gsems, wsems = (b0, b1), (g0, g1), (w0, w1)
        pltpu.sync_copy(idx_hbm.at[pl.ds(base_w * l, b_per_w * l)], idx_v)

        gload = [None, None]
        write = [[], []]

        def fire(c):
            bb = c % 2
            gload[bb] = [pltpu.async_copy(
                table_hbm.at[idx_v.at[pl.ds(c * rows_per_c + g * GW, GW)]],
                bufs[bb].at[pl.ds(g * GW, GW)],
                gsems[bb],
            ) for g in range(n_g)]

        fire(0)
        for c in range(n_chunks):
            bb = c % 2
            if c + 1 < n_chunks:
                nb = (c + 1) % 2
                for w in write[nb]:
                    w.wait()
                write[nb] = []
                fire(c + 1)
            for h in gload[bb]:
                h.wait()
            row0 = (base_w + c * NB) * l
            write[bb] = [pltpu.async_copy(
                bufs[bb].at[pl.ds(j * l, l)],
                out_hbm.at[pl.ds(row0 + j * l, l), pl.ds(0, D)],
                wsems[bb],
            ) for j in range(NB)]
        for ws in write:
            for w in ws:
                w.wait()

    return k


def _concat_body(gath_ref, other_ref, out_ref):
    bm, l, _ = out_ref.shape
    g = gath_ref[...].reshape(bm * l, 2 * D)
    out_ref[:, :, :D] = g[:, :D].reshape(bm, l, D)
    out_ref[:, :, D:] = other_ref[...]


def _make_concat(b: int, l: int, bm: int):
    return pl.pallas_call(
        _concat_body,
        grid=(b // bm,),
        in_specs=[
            pl.BlockSpec((bm * l * 2 * D,), lambda i: (i,)),
            pl.BlockSpec((bm, l, D), lambda i: (i, 0, 0)),
        ],
        out_specs=pl.BlockSpec((bm, l, 2 * D), lambda i: (i, 0, 0)),
        out_shape=jax.ShapeDtypeStruct((b, l, 2 * D), jnp.float32),
    )


def kernel(indices, other_features, table):
    b, l = indices.shape
    idx_i32 = indices.reshape(b * l).astype(jnp.int32)
    gath = _make_gather(b, l)(idx_i32, table)
    gath1 = gath.reshape(b * l * 2 * D)
    return _make_concat(b, l, 64)(gath1, other_features)


# speedup vs baseline: 9.2466x; 1.7254x over previous
import functools

import jax
import jax.numpy as jnp
from jax import lax
from jax.experimental import pallas as pl
from jax.experimental.pallas import tpu as pltpu
from jax.experimental.pallas import tpu_sc as plsc

D = 64
NC, NS = 2, 16
NW = NC * NS
GW = 80
CHUNK = 640


def _make_gather(b: int, l: int):
    b_blk = b // NW
    LC = 5
    n_chunks = l // LC

    mesh = plsc.VectorSubcoreMesh(core_axis_name="c", subcore_axis_name="s")

    @functools.partial(
        pl.kernel,
        mesh=mesh,
        compiler_params=pltpu.CompilerParams(use_tc_tiling_on_sc=False),
        out_type=jax.ShapeDtypeStruct((l, b, 2 * D), jnp.float32),
        scratch_types=[
            pltpu.VMEM((l, b_blk), jnp.int32),
            pltpu.VMEM((LC, b_blk, D), jnp.float32),
            pltpu.VMEM((LC, b_blk, D), jnp.float32),
            pltpu.SemaphoreType.DMA,
            pltpu.SemaphoreType.DMA,
            pltpu.SemaphoreType.DMA,
            pltpu.SemaphoreType.DMA,
        ],
    )
    def k(idx_hbm, table_hbm, out_hbm, idx_v, b0, b1, g0, g1, w0, w1):
        wid = lax.axis_index("s") * NC + lax.axis_index("c")
        wb = wid * b_blk
        bufs, gsems, wsems = (b0, b1), (g0, g1), (w0, w1)
        pltpu.sync_copy(idx_hbm.at[pl.ds(0, l), pl.ds(wb, b_blk)], idx_v)

        gload = [None, None]
        write = [None, None]

        def fire(c):
            bb = c % 2
            gload[bb] = [pltpu.async_copy(
                table_hbm.at[idx_v.at[c * LC + j]],
                bufs[bb].at[j],
                gsems[bb],
            ) for j in range(LC)]

        fire(0)
        for c in range(n_chunks):
            bb = c % 2
            if c + 1 < n_chunks:
                nb = (c + 1) % 2
                if write[nb] is not None:
                    write[nb].wait()
                fire(c + 1)
            for h in gload[bb]:
                h.wait()
            write[bb] = pltpu.async_copy(
                bufs[bb],
                out_hbm.at[pl.ds(c * LC, LC), pl.ds(wb, b_blk), pl.ds(0, D)],
                wsems[bb])
        for w in write:
            if w is not None:
                w.wait()

    return k


def _concat_body(gath_ref, other_ref, out_ref):
    l, bm, _ = out_ref.shape
    out_ref[:, :, :D] = gath_ref[:, :, :D]
    out_ref[:, :, D:] = jnp.swapaxes(other_ref[...], 1, 2)


def _make_concat(b: int, l: int, bm: int):
    return pl.pallas_call(
        _concat_body,
        grid=(b // bm,),
        in_specs=[
            pl.BlockSpec((l, bm, 2 * D), lambda i: (0, i, 0)),
            pl.BlockSpec((l, D, bm), lambda i: (0, 0, i)),
        ],
        out_specs=pl.BlockSpec((l, bm, 2 * D), lambda i: (0, i, 0)),
        out_shape=jax.ShapeDtypeStruct((l, b, 2 * D), jnp.float32),
    )


def kernel(indices, other_features, table):
    b, l = indices.shape
    idx_lm = indices.transpose(1, 0).astype(jnp.int32)
    gath3 = _make_gather(b, l)(idx_lm, table)
    other_t = other_features.transpose(1, 2, 0)
    out_t = _make_concat(b, l, 128)(gath3, other_t)
    return out_t.transpose(1, 0, 2)


# speedup vs baseline: 9.4307x; 1.0199x over previous
import functools

import jax
import jax.numpy as jnp
from jax import lax
from jax.experimental import pallas as pl
from jax.experimental.pallas import tpu as pltpu
from jax.experimental.pallas import tpu_sc as plsc

D = 64
NC, NS = 2, 16
NW = NC * NS
GW = 80
CHUNK = 640


def _make_gather(b: int, l: int):
    b_blk = b // NW
    LC = 5
    n_chunks = l // LC

    mesh = plsc.VectorSubcoreMesh(core_axis_name="c", subcore_axis_name="s")

    @functools.partial(
        pl.kernel,
        mesh=mesh,
        compiler_params=pltpu.CompilerParams(use_tc_tiling_on_sc=False),
        out_type=jax.ShapeDtypeStruct((l, b, 2 * D), jnp.float32),
        scratch_types=[
            pltpu.VMEM((l, b_blk), jnp.int32),
            pltpu.VMEM((LC, b_blk, D), jnp.float32),
            pltpu.VMEM((LC, b_blk, D), jnp.float32),
            pltpu.SemaphoreType.DMA,
            pltpu.SemaphoreType.DMA,
            pltpu.SemaphoreType.DMA,
            pltpu.SemaphoreType.DMA,
        ],
    )
    def k(idx_hbm, table_hbm, out_hbm, idx_v, b0, b1, g0, g1, w0, w1):
        wid = lax.axis_index("s") * NC + lax.axis_index("c")
        wb = wid * b_blk
        bufs, gsems, wsems = (b0, b1), (g0, g1), (w0, w1)
        pltpu.sync_copy(idx_hbm.at[pl.ds(0, l), pl.ds(wb, b_blk)], idx_v)

        gload = [None, None]
        write = [None, None]

        def fire(c):
            bb = c % 2
            gload[bb] = [pltpu.async_copy(
                table_hbm.at[idx_v.at[c * LC + j]],
                bufs[bb].at[j],
                gsems[bb],
            ) for j in range(LC)]

        fire(0)
        for c in range(n_chunks):
            bb = c % 2
            if c + 1 < n_chunks:
                nb = (c + 1) % 2
                if write[nb] is not None:
                    write[nb].wait()
                fire(c + 1)
            for h in gload[bb]:
                h.wait()
            write[bb] = pltpu.async_copy(
                bufs[bb],
                out_hbm.at[pl.ds(c * LC, LC), pl.ds(wb, b_blk), pl.ds(0, D)],
                wsems[bb])
        for w in write:
            if w is not None:
                w.wait()

    return k


def _concat_body(gath_ref, other_ref, out_ref):
    l, bm, _ = out_ref.shape
    out_ref[:, :, :D] = gath_ref[:, :, :D]
    out_ref[:, :, D:] = jnp.swapaxes(other_ref[...], 1, 2)


def _make_concat(b: int, l: int, bm: int):
    return pl.pallas_call(
        _concat_body,
        grid=(b // bm,),
        in_specs=[
            pl.BlockSpec((l, bm, 2 * D), lambda i: (0, i, 0)),
            pl.BlockSpec((l, D, bm), lambda i: (0, 0, i)),
        ],
        out_specs=pl.BlockSpec((l, bm, 2 * D), lambda i: (0, i, 0)),
        out_shape=jax.ShapeDtypeStruct((l, b, 2 * D), jnp.float32),
    )


def kernel(indices, other_features, table):
    b, l = indices.shape
    idx_lm = indices.transpose(1, 0).astype(jnp.int32)
    gath3 = _make_gather(b, l)(idx_lm, table)
    other_t = other_features.transpose(1, 2, 0)
    out_t = _make_concat(b, l, 256)(gath3, other_t)
    return out_t.transpose(1, 0, 2)
